# baseline (speedup 1.0000x reference)
import functools

import jax
import jax.numpy as jnp
from jax import lax
from jax.experimental import pallas as pl
from jax.experimental.pallas import tpu as pltpu
from jax.experimental.pallas import tpu_sc as plsc


_NC = 2
_NS = 16
_NW = _NC * _NS
_C = 7


_D = 16


def _sc_gather_body(b_per_w, chunk, table_hbm, idx_hbm, out_hbm, idx_c,
                    rows_v, pack_v, sem):
    wid = lax.axis_index("s") * _NC + lax.axis_index("c")
    base = wid * b_per_w

    def chunk_body(i, carry):
        off = i * chunk
        pltpu.sync_copy(idx_hbm.at[pl.ds(base + off, chunk)], idx_c)
        pltpu.async_copy(table_hbm.at[idx_c], rows_v, sem).wait()

        def pack_body(j, carry2):
            for k in range(8):
                pack_v[j, pl.ds(k * _D, _D)] = rows_v[j * 8 + k]
            return carry2

        lax.fori_loop(0, chunk // 8, pack_body, 0)
        pltpu.sync_copy(
            pack_v, out_hbm.at[pl.ds((base + off) // 8, chunk // 8)])
        return carry

    lax.fori_loop(0, b_per_w // chunk, chunk_body, 0)


def _gather_call(table, idx):
    b = idx.shape[0]
    assert b % (16 * _NW) == 0
    b_per_w = b // _NW
    chunk = 2000
    assert b_per_w % chunk == 0 and chunk % 16 == 0
    mesh = plsc.VectorSubcoreMesh(core_axis_name="c", subcore_axis_name="s",
                                  num_cores=_NC)
    f = pl.kernel(
        functools.partial(_sc_gather_body, b_per_w, chunk),
        out_type=jax.ShapeDtypeStruct((b // 8, 128), jnp.float32),
        mesh=mesh,
        scratch_types=[
            pltpu.VMEM((chunk,), jnp.int32),
            pltpu.VMEM((chunk, _D), jnp.float32),
            pltpu.VMEM((chunk // 8, 128), jnp.float32),
            pltpu.SemaphoreType.DMA,
        ],
        compiler_params=pltpu.CompilerParams(use_tc_tiling_on_sc=False),
    )
    return f(table, idx)


def _qmul(a, b):
    x1, y1, z1, w1 = a
    x2, y2, z2, w2 = b
    return (
        w1 * x2 + x1 * w2 + y1 * z2 - z1 * y2,
        w1 * y2 - x1 * z2 + y1 * w2 + z1 * x2,
        w1 * z2 + x1 * y2 - y1 * x2 + z1 * w2,
        w1 * w2 - x1 * x2 - y1 * y2 - z1 * z2,
    )


def _cross(a, b):
    a1, a2, a3 = a
    b1, b2, b3 = b
    return (a2 * b3 - a3 * b2, a3 * b1 - a1 * b3, a1 * b2 - a2 * b1)


def _qrot(q, v):
    x, y, z, w = q
    qv = (x, y, z)
    t = _cross(qv, v)
    t = (2.0 * t[0], 2.0 * t[1], 2.0 * t[2])
    c = _cross(qv, t)
    return (v[0] + w * t[0] + c[0], v[1] + w * t[1] + c[1], v[2] + w * t[2] + c[2])


def _so3_log_parts(q):
    x, y, z, w = q
    n2 = x * x + y * y + z * z
    n = jnp.sqrt(n2 + 1e-12)
    angle = 2.0 * jnp.arctan2(n, w)
    s = angle / n
    return (x * s, y * s, z * s), n2, n, w


def _pg_math(pt, pq, t1, q1, t2, q2):
    qa = (-pq[0], -pq[1], -pq[2], pq[3])
    ra = _qrot(qa, pt)
    ta = (-ra[0], -ra[1], -ra[2])
    qb = (-q1[0], -q1[1], -q1[2], q1[3])
    rb = _qrot(qb, t1)
    tb = (-rb[0], -rb[1], -rb[2])
    rab = _qrot(qa, tb)
    tc = (ta[0] + rab[0], ta[1] + rab[1], ta[2] + rab[2])
    qc = _qmul(qa, qb)
    rc2 = _qrot(qc, t2)
    te = (tc[0] + rc2[0], tc[1] + rc2[1], tc[2] + rc2[2])
    qe = _qmul(qc, q2)
    phi, _, n, w = _so3_log_parts(qe)
    theta2 = phi[0] * phi[0] + phi[1] * phi[1] + phi[2] * phi[2]
    theta = jnp.sqrt(theta2 + 1e-12)
    small = theta < 1e-3
    ts = jnp.where(small, 1.0, theta)
    coef = jnp.where(small, 1.0 / 12.0, (1.0 - ts * (w / (2.0 * n))) / (ts * ts))
    pxt = _cross(phi, te)
    cpp = _cross(phi, pxt)
    tau = (
        te[0] - 0.5 * pxt[0] + coef * cpp[0],
        te[1] - 0.5 * pxt[1] + coef * cpp[1],
        te[2] - 0.5 * pxt[2] + coef * cpp[2],
    )
    return tau, phi


def _tc_pg_body(g_ref, o_ref):
    rb = g_ref.shape[0]
    t = g_ref[...].T
    t3 = t.reshape(2, 64, rb)

    def comp(slot, c):
        return t3[:, 16 * slot + c, :]

    t1 = (comp(0, 0), comp(0, 1), comp(0, 2))
    q1 = (comp(0, 3), comp(0, 4), comp(0, 5), comp(0, 6))
    t2 = (comp(1, 0), comp(1, 1), comp(1, 2))
    q2 = (comp(1, 3), comp(1, 4), comp(1, 5), comp(1, 6))
    pt = (comp(2, 0), comp(2, 1), comp(2, 2))
    pq = (comp(2, 3), comp(2, 4), comp(2, 5), comp(2, 6))
    tau, phi = _pg_math(pt, pq, t1, q1, t2, q2)
    z = jnp.zeros((2, rb), jnp.float32)
    cols = [tau[0], tau[1], tau[2], phi[0], phi[1], phi[2]] + [z] * 58
    o3 = jnp.stack(cols, axis=1)
    o_ref[...] = o3.reshape(128, rb).T


def _tc_pg_call(gview):
    r = gview.shape[0]
    rb = 640
    assert r % rb == 0
    return pl.pallas_call(
        _tc_pg_body,
        grid=(r // rb,),
        in_specs=[pl.BlockSpec((rb, 128), lambda i: (i, 0))],
        out_specs=pl.BlockSpec((rb, 128), lambda i: (i, 0)),
        out_shape=jax.ShapeDtypeStruct((r, 128), jnp.float32),
    )(gview)


def _tc_chain_body(n1_ref, n2_ref, dr_ref, dv_ref, dtr_ref, dt_ref, v1_ref,
                   v2_ref, adj_ref, rot_ref, tv_ref):
    q1 = (n1_ref[3], n1_ref[4], n1_ref[5], n1_ref[6])
    q2 = (n2_ref[3], n2_ref[4], n2_ref[5], n2_ref[6])
    dr = (dr_ref[0], dr_ref[1], dr_ref[2], dr_ref[3])
    for k in range(3):
        adj_ref[k] = dv_ref[k] - (v2_ref[k] - v1_ref[k])
    a = _qmul((-dr[0], -dr[1], -dr[2], dr[3]), (-q1[0], -q1[1], -q1[2], q1[3]))
    rerr = _qmul(a, q2)
    phi, _, _, _ = _so3_log_parts(rerr)
    rot_ref[0] = phi[0]
    rot_ref[1] = phi[1]
    rot_ref[2] = phi[2]
    dt = dt_ref[0]
    for k in range(3):
        tv_ref[k] = (n2_ref[k] - n1_ref[k]) - (v1_ref[k] * dt + dtr_ref[k])


def _tc_chain_call(n1, n2, dr, dv, dtr, dt, v1, v2):
    nb = n1.shape[1]
    shp = lambda c: jax.ShapeDtypeStruct((c, nb, 128), jnp.float32)
    return pl.pallas_call(
        _tc_chain_body,
        out_shape=(shp(3), shp(3), shp(3)),
    )(n1, n2, dr, dv, dtr, dt, v1, v2)


def _to_soa(a, rows_pad):
    m, c = a.shape
    ap = jnp.pad(a, ((0, rows_pad - m), (0, 0)))
    return ap.T.reshape(c, rows_pad // 128, 128)


def kernel(edges, poses, imu_drots, imu_dtrans, imu_dvels, dts, nodes, vels):
    e = edges.shape[0]
    n = nodes.shape[0]

    nodes8 = jnp.pad(nodes, ((0, 0), (0, _D - nodes.shape[1])))
    poses8 = jnp.pad(poses, ((0, 0), (0, _D - poses.shape[1])))
    table = jnp.concatenate([nodes8, poses8], axis=0)
    pose_row = n + jnp.arange(e, dtype=jnp.int32)
    idx = jnp.stack([edges[:, 0], edges[:, 1], pose_row, pose_row],
                    axis=1).reshape(4 * e)
    table, idx = jax.lax.optimization_barrier((table, idx))
    af = _gather_call(table, idx)

    out = _tc_pg_call(af)
    pgerr = out.reshape(e, 64)[:, :6]

    m = n - 1
    mp = ((m + 128 * 8 - 1) // (128 * 8)) * (128 * 8)
    n1 = _to_soa(nodes[:-1], mp)
    n2 = _to_soa(nodes[1:], mp)
    dr = _to_soa(imu_drots, mp)
    dv = _to_soa(imu_dvels, mp)
    dtr = _to_soa(imu_dtrans, mp)
    dt = _to_soa(dts, mp)
    v1 = _to_soa(vels[:-1], mp)
    v2 = _to_soa(vels[1:], mp)
    adj, rot, tv = _tc_chain_call(n1, n2, dr, dv, dtr, dt, v1, v2)
    adjvelerr = adj.reshape(3, mp).T[:m]
    imuroterr = rot.reshape(3, mp).T[:m]
    transvelerr = tv.reshape(3, mp).T[:m]

    return (pgerr, adjvelerr, imuroterr, transvelerr)


# speedup vs baseline: 1.2039x; 1.2039x over previous
import functools

import jax
import jax.numpy as jnp
from jax import lax
from jax.experimental import pallas as pl
from jax.experimental.pallas import tpu as pltpu
from jax.experimental.pallas import tpu_sc as plsc


_NC = 2
_NS = 16
_NW = _NC * _NS
_C = 7


_D = 8


def _sc_gather_body(b_per_w, chunk, table_hbm, idx_hbm, out_hbm, idx_c,
                    rows_v, sem):
    wid = lax.axis_index("s") * _NC + lax.axis_index("c")
    base = wid * b_per_w

    def chunk_body(i, carry):
        off = i * chunk
        pltpu.sync_copy(idx_hbm.at[pl.ds(base + off, chunk)], idx_c)
        pltpu.async_copy(table_hbm.at[idx_c], rows_v, sem).wait()
        pltpu.sync_copy(rows_v, out_hbm.at[pl.ds(base + off, chunk)])
        return carry

    lax.fori_loop(0, b_per_w // chunk, chunk_body, 0)


def _gather_call(table, idx):
    b = idx.shape[0]
    assert b % (16 * _NW) == 0
    b_per_w = b // _NW
    chunk = 2000
    assert b_per_w % chunk == 0 and chunk % 16 == 0
    mesh = plsc.VectorSubcoreMesh(core_axis_name="c", subcore_axis_name="s",
                                  num_cores=_NC)
    f = pl.kernel(
        functools.partial(_sc_gather_body, b_per_w, chunk),
        out_type=jax.ShapeDtypeStruct((b, _D), jnp.float32),
        mesh=mesh,
        scratch_types=[
            pltpu.VMEM((chunk,), jnp.int32),
            pltpu.VMEM((chunk, _D), jnp.float32),
            pltpu.SemaphoreType.DMA,
        ],
        compiler_params=pltpu.CompilerParams(use_tc_tiling_on_sc=False),
    )
    return f(table, idx)


def _qmul(a, b):
    x1, y1, z1, w1 = a
    x2, y2, z2, w2 = b
    return (
        w1 * x2 + x1 * w2 + y1 * z2 - z1 * y2,
        w1 * y2 - x1 * z2 + y1 * w2 + z1 * x2,
        w1 * z2 + x1 * y2 - y1 * x2 + z1 * w2,
        w1 * w2 - x1 * x2 - y1 * y2 - z1 * z2,
    )


def _cross(a, b):
    a1, a2, a3 = a
    b1, b2, b3 = b
    return (a2 * b3 - a3 * b2, a3 * b1 - a1 * b3, a1 * b2 - a2 * b1)


def _qrot(q, v):
    x, y, z, w = q
    qv = (x, y, z)
    t = _cross(qv, v)
    t = (2.0 * t[0], 2.0 * t[1], 2.0 * t[2])
    c = _cross(qv, t)
    return (v[0] + w * t[0] + c[0], v[1] + w * t[1] + c[1], v[2] + w * t[2] + c[2])


def _so3_log_parts(q):
    x, y, z, w = q
    n2 = x * x + y * y + z * z
    n = jnp.sqrt(n2 + 1e-12)
    angle = 2.0 * jnp.arctan2(n, w)
    s = angle / n
    return (x * s, y * s, z * s), n2, n, w


def _pg_math(pt, pq, t1, q1, t2, q2):
    qa = (-pq[0], -pq[1], -pq[2], pq[3])
    ra = _qrot(qa, pt)
    ta = (-ra[0], -ra[1], -ra[2])
    qb = (-q1[0], -q1[1], -q1[2], q1[3])
    rb = _qrot(qb, t1)
    tb = (-rb[0], -rb[1], -rb[2])
    rab = _qrot(qa, tb)
    tc = (ta[0] + rab[0], ta[1] + rab[1], ta[2] + rab[2])
    qc = _qmul(qa, qb)
    rc2 = _qrot(qc, t2)
    te = (tc[0] + rc2[0], tc[1] + rc2[1], tc[2] + rc2[2])
    qe = _qmul(qc, q2)
    phi, _, n, w = _so3_log_parts(qe)
    theta2 = phi[0] * phi[0] + phi[1] * phi[1] + phi[2] * phi[2]
    theta = jnp.sqrt(theta2 + 1e-12)
    small = theta < 1e-3
    ts = jnp.where(small, 1.0, theta)
    coef = jnp.where(small, 1.0 / 12.0, (1.0 - ts * (w / (2.0 * n))) / (ts * ts))
    pxt = _cross(phi, te)
    cpp = _cross(phi, pxt)
    tau = (
        te[0] - 0.5 * pxt[0] + coef * cpp[0],
        te[1] - 0.5 * pxt[1] + coef * cpp[1],
        te[2] - 0.5 * pxt[2] + coef * cpp[2],
    )
    return tau, phi


def _tc_pg_body(g_ref, o_ref):
    rb = g_ref.shape[0]
    t = g_ref[...].T
    t3 = t.reshape(4, 32, rb)

    def comp(slot, c):
        return t3[:, 8 * slot + c, :]

    t1 = (comp(0, 0), comp(0, 1), comp(0, 2))
    q1 = (comp(0, 3), comp(0, 4), comp(0, 5), comp(0, 6))
    t2 = (comp(1, 0), comp(1, 1), comp(1, 2))
    q2 = (comp(1, 3), comp(1, 4), comp(1, 5), comp(1, 6))
    pt = (comp(2, 0), comp(2, 1), comp(2, 2))
    pq = (comp(2, 3), comp(2, 4), comp(2, 5), comp(2, 6))
    tau, phi = _pg_math(pt, pq, t1, q1, t2, q2)
    z = jnp.zeros((4, rb), jnp.float32)
    cols = [tau[0], tau[1], tau[2], phi[0], phi[1], phi[2]] + [z] * 26
    o3 = jnp.stack(cols, axis=1)
    o_ref[...] = o3.reshape(128, rb).T


def _tc_pg_call(gview):
    r = gview.shape[0]
    rb = 640
    assert r % rb == 0
    return pl.pallas_call(
        _tc_pg_body,
        grid=(r // rb,),
        in_specs=[pl.BlockSpec((rb, 128), lambda i: (i, 0))],
        out_specs=pl.BlockSpec((rb, 128), lambda i: (i, 0)),
        out_shape=jax.ShapeDtypeStruct((r, 128), jnp.float32),
    )(gview)


def _tc_chain_body(n1_ref, n2_ref, dr_ref, dv_ref, dtr_ref, dt_ref, v1_ref,
                   v2_ref, adj_ref, rot_ref, tv_ref):
    q1 = (n1_ref[3], n1_ref[4], n1_ref[5], n1_ref[6])
    q2 = (n2_ref[3], n2_ref[4], n2_ref[5], n2_ref[6])
    dr = (dr_ref[0], dr_ref[1], dr_ref[2], dr_ref[3])
    for k in range(3):
        adj_ref[k] = dv_ref[k] - (v2_ref[k] - v1_ref[k])
    a = _qmul((-dr[0], -dr[1], -dr[2], dr[3]), (-q1[0], -q1[1], -q1[2], q1[3]))
    rerr = _qmul(a, q2)
    phi, _, _, _ = _so3_log_parts(rerr)
    rot_ref[0] = phi[0]
    rot_ref[1] = phi[1]
    rot_ref[2] = phi[2]
    dt = dt_ref[0]
    for k in range(3):
        tv_ref[k] = (n2_ref[k] - n1_ref[k]) - (v1_ref[k] * dt + dtr_ref[k])


def _tc_chain_call(n1, n2, dr, dv, dtr, dt, v1, v2):
    nb = n1.shape[1]
    shp = lambda c: jax.ShapeDtypeStruct((c, nb, 128), jnp.float32)
    return pl.pallas_call(
        _tc_chain_body,
        out_shape=(shp(3), shp(3), shp(3)),
    )(n1, n2, dr, dv, dtr, dt, v1, v2)


def _to_soa(a, rows_pad):
    m, c = a.shape
    ap = jnp.pad(a, ((0, rows_pad - m), (0, 0)))
    return ap.T.reshape(c, rows_pad // 128, 128)


def kernel(edges, poses, imu_drots, imu_dtrans, imu_dvels, dts, nodes, vels):
    e = edges.shape[0]
    n = nodes.shape[0]

    nodes8 = jnp.pad(nodes, ((0, 0), (0, _D - nodes.shape[1])))
    poses8 = jnp.pad(poses, ((0, 0), (0, _D - poses.shape[1])))
    table = jnp.concatenate([nodes8, poses8], axis=0)
    pose_row = n + jnp.arange(e, dtype=jnp.int32)
    idx = jnp.stack([edges[:, 0], edges[:, 1], pose_row, pose_row],
                    axis=1).reshape(4 * e)
    af = _gather_call(table, idx)

    gview = af.reshape(4 * e * _D // 128, 128)
    out = _tc_pg_call(gview)
    pgerr = out.reshape(e, 32)[:, :6]

    m = n - 1
    mp = ((m + 128 * 8 - 1) // (128 * 8)) * (128 * 8)
    n1 = _to_soa(nodes[:-1], mp)
    n2 = _to_soa(nodes[1:], mp)
    dr = _to_soa(imu_drots, mp)
    dv = _to_soa(imu_dvels, mp)
    dtr = _to_soa(imu_dtrans, mp)
    dt = _to_soa(dts, mp)
    v1 = _to_soa(vels[:-1], mp)
    v2 = _to_soa(vels[1:], mp)
    adj, rot, tv = _tc_chain_call(n1, n2, dr, dv, dtr, dt, v1, v2)
    adjvelerr = adj.reshape(3, mp).T[:m]
    imuroterr = rot.reshape(3, mp).T[:m]
    transvelerr = tv.reshape(3, mp).T[:m]

    return (pgerr, adjvelerr, imuroterr, transvelerr)


# speedup vs baseline: 1.2465x; 1.0354x over previous
import functools

import jax
import jax.numpy as jnp
from jax import lax
from jax.experimental import pallas as pl
from jax.experimental.pallas import tpu as pltpu
from jax.experimental.pallas import tpu_sc as plsc


_NC = 2
_NS = 16
_NW = _NC * _NS
_D = 8


def _sc_gather_body(b_per_w, chunk, table_hbm, idx_hbm, out_hbm, idx_a,
                    idx_b, rows_a, rows_b, sem_a, sem_b):
    wid = lax.axis_index("s") * _NC + lax.axis_index("c")
    base = wid * b_per_w

    def pair_body(i, carry):
        off_a = (2 * i) * chunk
        off_b = (2 * i + 1) * chunk
        pltpu.sync_copy(idx_hbm.at[pl.ds(base + off_a, chunk)], idx_a)
        cp_a = pltpu.async_copy(table_hbm.at[idx_a], rows_a, sem_a)
        pltpu.sync_copy(idx_hbm.at[pl.ds(base + off_b, chunk)], idx_b)
        cp_b = pltpu.async_copy(table_hbm.at[idx_b], rows_b, sem_b)
        cp_a.wait()
        pltpu.sync_copy(rows_a, out_hbm.at[pl.ds(base + off_a, chunk)])
        cp_b.wait()
        pltpu.sync_copy(rows_b, out_hbm.at[pl.ds(base + off_b, chunk)])
        return carry

    lax.fori_loop(0, b_per_w // (2 * chunk), pair_body, 0)


def _gather_call(table, idx):
    b = idx.shape[0]
    assert b % (16 * _NW) == 0
    b_per_w = b // _NW
    chunk = 4000
    assert b_per_w % (2 * chunk) == 0 and chunk % 16 == 0
    mesh = plsc.VectorSubcoreMesh(core_axis_name="c", subcore_axis_name="s",
                                  num_cores=_NC)
    f = pl.kernel(
        functools.partial(_sc_gather_body, b_per_w, chunk),
        out_type=jax.ShapeDtypeStruct((b, _D), jnp.float32),
        mesh=mesh,
        scratch_types=[
            pltpu.VMEM((chunk,), jnp.int32),
            pltpu.VMEM((chunk,), jnp.int32),
            pltpu.VMEM((chunk, _D), jnp.float32),
            pltpu.VMEM((chunk, _D), jnp.float32),
            pltpu.SemaphoreType.DMA,
            pltpu.SemaphoreType.DMA,
        ],
        compiler_params=pltpu.CompilerParams(use_tc_tiling_on_sc=False),
    )
    return f(table, idx)


def _qmul(a, b):
    x1, y1, z1, w1 = a
    x2, y2, z2, w2 = b
    return (
        w1 * x2 + x1 * w2 + y1 * z2 - z1 * y2,
        w1 * y2 - x1 * z2 + y1 * w2 + z1 * x2,
        w1 * z2 + x1 * y2 - y1 * x2 + z1 * w2,
        w1 * w2 - x1 * x2 - y1 * y2 - z1 * z2,
    )


def _cross(a, b):
    a1, a2, a3 = a
    b1, b2, b3 = b
    return (a2 * b3 - a3 * b2, a3 * b1 - a1 * b3, a1 * b2 - a2 * b1)


def _qrot(q, v):
    x, y, z, w = q
    qv = (x, y, z)
    t = _cross(qv, v)
    t = (2.0 * t[0], 2.0 * t[1], 2.0 * t[2])
    c = _cross(qv, t)
    return (v[0] + w * t[0] + c[0], v[1] + w * t[1] + c[1], v[2] + w * t[2] + c[2])


def _so3_log_parts(q):
    x, y, z, w = q
    n2 = x * x + y * y + z * z
    n = jnp.sqrt(n2 + 1e-12)
    angle = 2.0 * jnp.arctan2(n, w)
    s = angle / n
    return (x * s, y * s, z * s), n2, n, w


def _pg_math(pt, pq, t1, q1, t2, q2):
    qa = (-pq[0], -pq[1], -pq[2], pq[3])
    ra = _qrot(qa, pt)
    ta = (-ra[0], -ra[1], -ra[2])
    qb = (-q1[0], -q1[1], -q1[2], q1[3])
    rb = _qrot(qb, t1)
    tb = (-rb[0], -rb[1], -rb[2])
    rab = _qrot(qa, tb)
    tc = (ta[0] + rab[0], ta[1] + rab[1], ta[2] + rab[2])
    qc = _qmul(qa, qb)
    rc2 = _qrot(qc, t2)
    te = (tc[0] + rc2[0], tc[1] + rc2[1], tc[2] + rc2[2])
    qe = _qmul(qc, q2)
    phi, _, n, w = _so3_log_parts(qe)
    theta2 = phi[0] * phi[0] + phi[1] * phi[1] + phi[2] * phi[2]
    theta = jnp.sqrt(theta2 + 1e-12)
    small = theta < 1e-3
    ts = jnp.where(small, 1.0, theta)
    coef = jnp.where(small, 1.0 / 12.0, (1.0 - ts * (w / (2.0 * n))) / (ts * ts))
    pxt = _cross(phi, te)
    cpp = _cross(phi, pxt)
    tau = (
        te[0] - 0.5 * pxt[0] + coef * cpp[0],
        te[1] - 0.5 * pxt[1] + coef * cpp[1],
        te[2] - 0.5 * pxt[2] + coef * cpp[2],
    )
    return tau, phi


def _tc_pg_body(g_ref, o_ref):
    rb = g_ref.shape[0]
    t = g_ref[...].T
    t3 = t.reshape(4, 32, rb)

    def comp(slot, c):
        return t3[:, 8 * slot + c, :]

    t1 = (comp(0, 0), comp(0, 1), comp(0, 2))
    q1 = (comp(0, 3), comp(0, 4), comp(0, 5), comp(0, 6))
    t2 = (comp(1, 0), comp(1, 1), comp(1, 2))
    q2 = (comp(1, 3), comp(1, 4), comp(1, 5), comp(1, 6))
    pt = (comp(2, 0), comp(2, 1), comp(2, 2))
    pq = (comp(2, 3), comp(2, 4), comp(2, 5), comp(2, 6))
    tau, phi = _pg_math(pt, pq, t1, q1, t2, q2)
    z = jnp.zeros((4, rb), jnp.float32)
    cols = [tau[0], tau[1], tau[2], phi[0], phi[1], phi[2]] + [z] * 26
    o3 = jnp.stack(cols, axis=1)
    o_ref[...] = o3.reshape(128, rb).T


def _tc_pg_call(gview):
    r = gview.shape[0]
    rb = 640
    assert r % rb == 0
    return pl.pallas_call(
        _tc_pg_body,
        grid=(r // rb,),
        in_specs=[pl.BlockSpec((rb, 128), lambda i: (i, 0))],
        out_specs=pl.BlockSpec((rb, 128), lambda i: (i, 0)),
        out_shape=jax.ShapeDtypeStruct((r, 128), jnp.float32),
    )(gview)


def _tc_chain_body(n1_ref, n2_ref, dr_ref, dv_ref, dtr_ref, dt_ref, v1_ref,
                   v2_ref, adj_ref, rot_ref, tv_ref):
    q1 = (n1_ref[3], n1_ref[4], n1_ref[5], n1_ref[6])
    q2 = (n2_ref[3], n2_ref[4], n2_ref[5], n2_ref[6])
    dr = (dr_ref[0], dr_ref[1], dr_ref[2], dr_ref[3])
    for k in range(3):
        adj_ref[k] = dv_ref[k] - (v2_ref[k] - v1_ref[k])
    a = _qmul((-dr[0], -dr[1], -dr[2], dr[3]), (-q1[0], -q1[1], -q1[2], q1[3]))
    rerr = _qmul(a, q2)
    phi, _, _, _ = _so3_log_parts(rerr)
    rot_ref[0] = phi[0]
    rot_ref[1] = phi[1]
    rot_ref[2] = phi[2]
    dt = dt_ref[0]
    for k in range(3):
        tv_ref[k] = (n2_ref[k] - n1_ref[k]) - (v1_ref[k] * dt + dtr_ref[k])


def _tc_chain_call(n1, n2, dr, dv, dtr, dt, v1, v2):
    nb = n1.shape[1]
    shp = lambda c: jax.ShapeDtypeStruct((c, nb, 128), jnp.float32)
    return pl.pallas_call(
        _tc_chain_body,
        out_shape=(shp(3), shp(3), shp(3)),
    )(n1, n2, dr, dv, dtr, dt, v1, v2)


def _to_soa(a, rows_pad):
    m, c = a.shape
    ap = jnp.pad(a, ((0, rows_pad - m), (0, 0)))
    return ap.T.reshape(c, rows_pad // 128, 128)


def kernel(edges, poses, imu_drots, imu_dtrans, imu_dvels, dts, nodes, vels):
    e = edges.shape[0]
    n = nodes.shape[0]

    nodes8 = jnp.pad(nodes, ((0, 0), (0, _D - nodes.shape[1])))
    poses8 = jnp.pad(poses, ((0, 0), (0, _D - poses.shape[1])))
    table = jnp.concatenate([nodes8, poses8], axis=0)
    pose_row = n + jnp.arange(e, dtype=jnp.int32)
    idx = jnp.stack([edges[:, 0], edges[:, 1], pose_row, pose_row],
                    axis=1).reshape(4 * e)
    af = _gather_call(table, idx)

    gview = af.reshape(4 * e * _D // 128, 128)
    out = _tc_pg_call(gview)
    pgerr = out.reshape(e, 32)[:, :6]

    m = n - 1
    mp = ((m + 128 * 8 - 1) // (128 * 8)) * (128 * 8)
    n1 = _to_soa(nodes[:-1], mp)
    n2 = _to_soa(nodes[1:], mp)
    dr = _to_soa(imu_drots, mp)
    dv = _to_soa(imu_dvels, mp)
    dtr = _to_soa(imu_dtrans, mp)
    dt = _to_soa(dts, mp)
    v1 = _to_soa(vels[:-1], mp)
    v2 = _to_soa(vels[1:], mp)
    adj, rot, tv = _tc_chain_call(n1, n2, dr, dv, dtr, dt, v1, v2)
    adjvelerr = adj.reshape(3, mp).T[:m]
    imuroterr = rot.reshape(3, mp).T[:m]
    transvelerr = tv.reshape(3, mp).T[:m]

    return (pgerr, adjvelerr, imuroterr, transvelerr)


# speedup vs baseline: 1.2478x; 1.0010x over previous
import functools

import jax
import jax.numpy as jnp
from jax import lax
from jax.experimental import pallas as pl
from jax.experimental.pallas import tpu as pltpu
from jax.experimental.pallas import tpu_sc as plsc


_NC = 2
_NS = 16
_NW = _NC * _NS
_D = 8


def _sc_gather_body(b_per_w, chunk, table_hbm, idx_hbm, out_hbm, idx_a,
                    idx_b, rows_a, rows_b, sem_a, sem_b):
    wid = lax.axis_index("s") * _NC + lax.axis_index("c")
    base = wid * b_per_w

    def pair_body(i, carry):
        off_a = (2 * i) * chunk
        off_b = (2 * i + 1) * chunk
        pltpu.sync_copy(idx_hbm.at[pl.ds(base + off_a, chunk)], idx_a)
        cp_a = pltpu.async_copy(table_hbm.at[idx_a], rows_a, sem_a)
        pltpu.sync_copy(idx_hbm.at[pl.ds(base + off_b, chunk)], idx_b)
        cp_b = pltpu.async_copy(table_hbm.at[idx_b], rows_b, sem_b)
        cp_a.wait()
        pltpu.sync_copy(rows_a, out_hbm.at[pl.ds(base + off_a, chunk)])
        cp_b.wait()
        pltpu.sync_copy(rows_b, out_hbm.at[pl.ds(base + off_b, chunk)])
        return carry

    lax.fori_loop(0, b_per_w // (2 * chunk), pair_body, 0)


def _gather_call(table, idx):
    b = idx.shape[0]
    assert b % (16 * _NW) == 0
    b_per_w = b // _NW
    chunk = 5000
    assert b_per_w % (2 * chunk) == 0 and chunk % 8 == 0
    mesh = plsc.VectorSubcoreMesh(core_axis_name="c", subcore_axis_name="s",
                                  num_cores=_NC)
    f = pl.kernel(
        functools.partial(_sc_gather_body, b_per_w, chunk),
        out_type=jax.ShapeDtypeStruct((b, _D), jnp.float32),
        mesh=mesh,
        scratch_types=[
            pltpu.VMEM((chunk,), jnp.int32),
            pltpu.VMEM((chunk,), jnp.int32),
            pltpu.VMEM((chunk, _D), jnp.float32),
            pltpu.VMEM((chunk, _D), jnp.float32),
            pltpu.SemaphoreType.DMA,
            pltpu.SemaphoreType.DMA,
        ],
        compiler_params=pltpu.CompilerParams(use_tc_tiling_on_sc=False),
    )
    return f(table, idx)


def _qmul(a, b):
    x1, y1, z1, w1 = a
    x2, y2, z2, w2 = b
    return (
        w1 * x2 + x1 * w2 + y1 * z2 - z1 * y2,
        w1 * y2 - x1 * z2 + y1 * w2 + z1 * x2,
        w1 * z2 + x1 * y2 - y1 * x2 + z1 * w2,
        w1 * w2 - x1 * x2 - y1 * y2 - z1 * z2,
    )


def _cross(a, b):
    a1, a2, a3 = a
    b1, b2, b3 = b
    return (a2 * b3 - a3 * b2, a3 * b1 - a1 * b3, a1 * b2 - a2 * b1)


def _qrot(q, v):
    x, y, z, w = q
    qv = (x, y, z)
    t = _cross(qv, v)
    t = (2.0 * t[0], 2.0 * t[1], 2.0 * t[2])
    c = _cross(qv, t)
    return (v[0] + w * t[0] + c[0], v[1] + w * t[1] + c[1], v[2] + w * t[2] + c[2])


def _so3_log_parts(q):
    x, y, z, w = q
    n2 = x * x + y * y + z * z
    n = jnp.sqrt(n2 + 1e-12)
    angle = 2.0 * jnp.arctan2(n, w)
    s = angle / n
    return (x * s, y * s, z * s), n2, n, w


def _pg_math(pt, pq, t1, q1, t2, q2):
    qa = (-pq[0], -pq[1], -pq[2], pq[3])
    ra = _qrot(qa, pt)
    ta = (-ra[0], -ra[1], -ra[2])
    qb = (-q1[0], -q1[1], -q1[2], q1[3])
    rb = _qrot(qb, t1)
    tb = (-rb[0], -rb[1], -rb[2])
    rab = _qrot(qa, tb)
    tc = (ta[0] + rab[0], ta[1] + rab[1], ta[2] + rab[2])
    qc = _qmul(qa, qb)
    rc2 = _qrot(qc, t2)
    te = (tc[0] + rc2[0], tc[1] + rc2[1], tc[2] + rc2[2])
    qe = _qmul(qc, q2)
    phi, _, n, w = _so3_log_parts(qe)
    theta2 = phi[0] * phi[0] + phi[1] * phi[1] + phi[2] * phi[2]
    theta = jnp.sqrt(theta2 + 1e-12)
    small = theta < 1e-3
    ts = jnp.where(small, 1.0, theta)
    coef = jnp.where(small, 1.0 / 12.0, (1.0 - ts * (w / (2.0 * n))) / (ts * ts))
    pxt = _cross(phi, te)
    cpp = _cross(phi, pxt)
    tau = (
        te[0] - 0.5 * pxt[0] + coef * cpp[0],
        te[1] - 0.5 * pxt[1] + coef * cpp[1],
        te[2] - 0.5 * pxt[2] + coef * cpp[2],
    )
    return tau, phi


def _tc_pg_body(g_ref, o_ref):
    rb = g_ref.shape[0]
    t = g_ref[...].T
    t3 = t.reshape(4, 32, rb)

    def comp(slot, c):
        return t3[:, 8 * slot + c, :]

    t1 = (comp(0, 0), comp(0, 1), comp(0, 2))
    q1 = (comp(0, 3), comp(0, 4), comp(0, 5), comp(0, 6))
    t2 = (comp(1, 0), comp(1, 1), comp(1, 2))
    q2 = (comp(1, 3), comp(1, 4), comp(1, 5), comp(1, 6))
    pt = (comp(2, 0), comp(2, 1), comp(2, 2))
    pq = (comp(2, 3), comp(2, 4), comp(2, 5), comp(2, 6))
    tau, phi = _pg_math(pt, pq, t1, q1, t2, q2)
    z = jnp.zeros((4, rb), jnp.float32)
    cols = [tau[0], tau[1], tau[2], phi[0], phi[1], phi[2]] + [z] * 26
    o3 = jnp.stack(cols, axis=1)
    o_ref[...] = o3.reshape(128, rb).T


def _tc_pg_call(gview):
    r = gview.shape[0]
    rb = 640
    assert r % rb == 0
    return pl.pallas_call(
        _tc_pg_body,
        grid=(r // rb,),
        in_specs=[pl.BlockSpec((rb, 128), lambda i: (i, 0))],
        out_specs=pl.BlockSpec((rb, 128), lambda i: (i, 0)),
        out_shape=jax.ShapeDtypeStruct((r, 128), jnp.float32),
    )(gview)


def _tc_chain_body(n1_ref, n2_ref, dr_ref, dv_ref, dtr_ref, dt_ref, v1_ref,
                   v2_ref, adj_ref, rot_ref, tv_ref):
    q1 = (n1_ref[3], n1_ref[4], n1_ref[5], n1_ref[6])
    q2 = (n2_ref[3], n2_ref[4], n2_ref[5], n2_ref[6])
    dr = (dr_ref[0], dr_ref[1], dr_ref[2], dr_ref[3])
    for k in range(3):
        adj_ref[k] = dv_ref[k] - (v2_ref[k] - v1_ref[k])
    a = _qmul((-dr[0], -dr[1], -dr[2], dr[3]), (-q1[0], -q1[1], -q1[2], q1[3]))
    rerr = _qmul(a, q2)
    phi, _, _, _ = _so3_log_parts(rerr)
    rot_ref[0] = phi[0]
    rot_ref[1] = phi[1]
    rot_ref[2] = phi[2]
    dt = dt_ref[0]
    for k in range(3):
        tv_ref[k] = (n2_ref[k] - n1_ref[k]) - (v1_ref[k] * dt + dtr_ref[k])


def _tc_chain_call(n1, n2, dr, dv, dtr, dt, v1, v2):
    nb = n1.shape[1]
    shp = lambda c: jax.ShapeDtypeStruct((c, nb, 128), jnp.float32)
    return pl.pallas_call(
        _tc_chain_body,
        out_shape=(shp(3), shp(3), shp(3)),
    )(n1, n2, dr, dv, dtr, dt, v1, v2)


def _to_soa(a, rows_pad):
    m, c = a.shape
    ap = jnp.pad(a, ((0, rows_pad - m), (0, 0)))
    return ap.T.reshape(c, rows_pad // 128, 128)


def kernel(edges, poses, imu_drots, imu_dtrans, imu_dvels, dts, nodes, vels):
    e = edges.shape[0]
    n = nodes.shape[0]

    nodes8 = jnp.pad(nodes, ((0, 0), (0, _D - nodes.shape[1])))
    poses8 = jnp.pad(poses, ((0, 0), (0, _D - poses.shape[1])))
    table = jnp.concatenate([nodes8, poses8], axis=0)
    pose_row = n + jnp.arange(e, dtype=jnp.int32)
    idx = jnp.stack([edges[:, 0], edges[:, 1], pose_row, pose_row],
                    axis=1).reshape(4 * e)
    af = _gather_call(table, idx)

    gview = af.reshape(4 * e * _D // 128, 128)
    out = _tc_pg_call(gview)
    pgerr = out.reshape(e, 32)[:, :6]

    m = n - 1
    mp = ((m + 128 * 8 - 1) // (128 * 8)) * (128 * 8)
    n1 = _to_soa(nodes[:-1], mp)
    n2 = _to_soa(nodes[1:], mp)
    dr = _to_soa(imu_drots, mp)
    dv = _to_soa(imu_dvels, mp)
    dtr = _to_soa(imu_dtrans, mp)
    dt = _to_soa(dts, mp)
    v1 = _to_soa(vels[:-1], mp)
    v2 = _to_soa(vels[1:], mp)
    adj, rot, tv = _tc_chain_call(n1, n2, dr, dv, dtr, dt, v1, v2)
    adjvelerr = adj.reshape(3, mp).T[:m]
    imuroterr = rot.reshape(3, mp).T[:m]
    transvelerr = tv.reshape(3, mp).T[:m]

    return (pgerr, adjvelerr, imuroterr, transvelerr)
